# Initial kernel scaffold; baseline (speedup 1.0000x reference)
import jax
import jax.numpy as jnp
from jax.experimental import pallas as pl


def kernel(indices, emb_table):
    raise NotImplementedError("write your pallas kernel here")


# speedup vs baseline: 1.8392x; 1.8392x over previous
import functools

import jax
import jax.numpy as jnp
from jax import lax
from jax.experimental import pallas as pl
from jax.experimental.pallas import tpu as pltpu
from jax.experimental.pallas import tpu_sc as plsc

VOCAB = 1000000
D = 64
B_TOTAL = 16384 * 50
NW = 32
B_PER_W = B_TOTAL // NW
CHUNK = 128
N_CHUNKS = B_PER_W // CHUNK


def _gather_body(idx_hbm, table_hbm, out_hbm, idx_v, rows_a, rows_b, sem_a, sem_b):
    wid = lax.axis_index("s") * 2 + lax.axis_index("c")
    base = wid * B_PER_W
    pltpu.sync_copy(idx_hbm.at[wid], idx_v)

    rows = (rows_a, rows_b)
    sems = (sem_a, sem_b)

    def start(j, slot):
        pltpu.async_copy(table_hbm.at[idx_v.at[j]], rows[slot], sems[slot])

    def drain(j, slot):
        pltpu.make_async_copy(table_hbm.at[idx_v.at[j]], rows[slot], sems[slot]).wait()
        pltpu.sync_copy(rows[slot], out_hbm.at[pl.ds(base + j * CHUNK, CHUNK)])

    start(0, 0)

    def loop(g, _):
        j = 2 * g
        start(j + 1, 1)
        drain(j, 0)
        start(j + 2, 0)
        drain(j + 1, 1)
        return 0

    lax.fori_loop(0, N_CHUNKS // 2 - 1, loop, 0)
    start(N_CHUNKS - 1, 1)
    drain(N_CHUNKS - 2, 0)
    drain(N_CHUNKS - 1, 1)


@jax.jit
def _gather(idx_grouped, emb_table):
    mesh = plsc.VectorSubcoreMesh(core_axis_name="c", subcore_axis_name="s")
    kfn = functools.partial(
        pl.kernel,
        mesh=mesh,
        out_type=jax.ShapeDtypeStruct((B_TOTAL, D), jnp.float32),
        scratch_types=[
            pltpu.VMEM((N_CHUNKS, CHUNK), jnp.int32),
            pltpu.VMEM((CHUNK, D), jnp.float32),
            pltpu.VMEM((CHUNK, D), jnp.float32),
            pltpu.SemaphoreType.DMA,
            pltpu.SemaphoreType.DMA,
        ],
        compiler_params=pltpu.CompilerParams(use_tc_tiling_on_sc=False),
    )(_gather_body)
    return kfn(idx_grouped, emb_table)


def kernel(indices, emb_table):
    idx_grouped = indices.reshape(NW, N_CHUNKS, CHUNK).astype(jnp.int32)
    out = _gather(idx_grouped, emb_table)
    return out.reshape(indices.shape[0], indices.shape[1], D)


# speedup vs baseline: 1.8737x; 1.0188x over previous
import functools

import jax
import jax.numpy as jnp
from jax import lax
from jax.experimental import pallas as pl
from jax.experimental.pallas import tpu as pltpu
from jax.experimental.pallas import tpu_sc as plsc

VOCAB = 1000000
D = 64
B_TOTAL = 16384 * 50
NW = 32
B_PER_W = B_TOTAL // NW
CHUNK = 128
N_CHUNKS = B_PER_W // CHUNK


NBUF = 8


def _gather_body(idx_hbm, table_hbm, out_hbm, idx_v, *rows_and_sems):
    rows = rows_and_sems[:NBUF]
    sems = rows_and_sems[NBUF:]
    wid = lax.axis_index("s") * 2 + lax.axis_index("c")
    base = wid * B_PER_W
    pltpu.sync_copy(idx_hbm.at[wid], idx_v)

    def start(j, slot):
        pltpu.async_copy(table_hbm.at[idx_v.at[j]], rows[slot], sems[slot])

    def drain(j, slot):
        pltpu.make_async_copy(table_hbm.at[idx_v.at[j]], rows[slot], sems[slot]).wait()
        pltpu.sync_copy(rows[slot], out_hbm.at[pl.ds(base + j * CHUNK, CHUNK)])

    for b in range(NBUF):
        start(b, b)

    def loop(g, _):
        j = g * NBUF
        for b in range(NBUF):
            drain(j + b, b)
            start(j + NBUF + b, b)
        return 0

    lax.fori_loop(0, N_CHUNKS // NBUF - 1, loop, 0)
    for b in range(NBUF):
        drain(N_CHUNKS - NBUF + b, b)


@jax.jit
def _gather(idx_grouped, emb_table):
    mesh = plsc.VectorSubcoreMesh(core_axis_name="c", subcore_axis_name="s")
    kfn = functools.partial(
        pl.kernel,
        mesh=mesh,
        out_type=jax.ShapeDtypeStruct((B_TOTAL, D), jnp.float32),
        scratch_types=(
            [pltpu.VMEM((N_CHUNKS, CHUNK), jnp.int32)]
            + [pltpu.VMEM((CHUNK, D), jnp.float32) for _ in range(NBUF)]
            + [pltpu.SemaphoreType.DMA for _ in range(NBUF)]
        ),
        compiler_params=pltpu.CompilerParams(use_tc_tiling_on_sc=False),
    )(_gather_body)
    return kfn(idx_grouped, emb_table)


def kernel(indices, emb_table):
    idx_grouped = indices.reshape(NW, N_CHUNKS, CHUNK).astype(jnp.int32)
    out = _gather(idx_grouped, emb_table)
    return out.reshape(indices.shape[0], indices.shape[1], D)
